# Initial kernel scaffold; baseline (speedup 1.0000x reference)
import jax
import jax.numpy as jnp
from jax.experimental import pallas as pl


def kernel(features, table, W1, b1, W2, b2):
    raise NotImplementedError("write your pallas kernel here")


# speedup vs baseline: 2.3203x; 2.3203x over previous
import functools

import jax
import jax.numpy as jnp
from jax import lax
from jax.experimental import pallas as pl
from jax.experimental.pallas import tpu as pltpu
from jax.experimental.pallas import tpu_sc as plsc


def _gather_pool_sc(flat_idx, table, batch, seq, emb):
    info = plsc.get_sparse_core_info()
    nc, ns = info.num_cores, info.num_subcores
    nw = nc * ns
    rows_w = batch // nw
    g_rows = 4
    g_idx = g_rows * seq
    n_groups = rows_w // g_rows
    mesh = plsc.VectorSubcoreMesh(core_axis_name="c", subcore_axis_name="s")

    @functools.partial(
        pl.kernel,
        out_type=jax.ShapeDtypeStruct((batch, emb), jnp.float32),
        mesh=mesh,
        scratch_types=[
            pltpu.VMEM((g_idx,), jnp.int32),
            pltpu.VMEM((g_idx,), jnp.int32),
            pltpu.VMEM((g_idx, emb), jnp.float32),
            pltpu.VMEM((g_idx, emb), jnp.float32),
            pltpu.VMEM((rows_w, emb), jnp.float32),
            pltpu.SemaphoreType.DMA,
            pltpu.SemaphoreType.DMA,
        ],
        compiler_params=pltpu.CompilerParams(use_tc_tiling_on_sc=False),
    )
    def k(idx_hbm, table_hbm, out_hbm, idx0, idx1, rows0, rows1, pooled,
          sem0, sem1):
        wid = lax.axis_index("s") * nc + lax.axis_index("c")
        base = wid * (rows_w * seq)
        idx_bufs = (idx0, idx1)
        row_bufs = (rows0, rows1)
        sems = (sem0, sem1)

        def start_gather(gi, b):
            pltpu.sync_copy(idx_hbm.at[pl.ds(base + gi * g_idx, g_idx)],
                            idx_bufs[b])
            pltpu.async_copy(table_hbm.at[idx_bufs[b]], row_bufs[b],
                             sems[b])

        def reduce_group(gi, b):
            rows = row_bufs[b]
            for r in range(g_rows):
                roff = r * seq

                def body(jj, carry):
                    a0, a1 = carry
                    j = roff + jj * 8
                    for u in range(8):
                        a0 = a0 + rows[j + u, pl.ds(0, 16)]
                        a1 = a1 + rows[j + u, pl.ds(16, 16)]
                    return (a0, a1)

                z = jnp.zeros((16,), jnp.float32)
                a0, a1 = lax.fori_loop(0, seq // 8, body, (z, z))
                row = gi * g_rows + r
                pooled[row, pl.ds(0, 16)] = a0
                pooled[row, pl.ds(16, 16)] = a1

        start_gather(0, 0)
        for gi in range(n_groups):
            b = gi % 2
            if gi + 1 < n_groups:
                start_gather(gi + 1, 1 - b)
            pltpu.make_async_copy(table_hbm.at[idx_bufs[b]], row_bufs[b],
                                  sems[b]).wait()
            reduce_group(gi, b)
        pltpu.sync_copy(pooled, out_hbm.at[pl.ds(wid * rows_w, rows_w)])

    return k(flat_idx, table)


def _mlp_tc(pooled, W1, b1, W2, b2, inv_seq):
    batch = pooled.shape[0]

    def body(p_ref, w1_ref, b1_ref, w2_ref, b2_ref, o_ref):
        x = p_ref[...] * inv_seq
        h = jnp.dot(x, w1_ref[...], precision=lax.Precision.HIGHEST)
        h = jnp.maximum(h + b1_ref[...], 0.0)
        o = jnp.dot(h, w2_ref[...], precision=lax.Precision.HIGHEST)
        o_ref[...] = jax.nn.sigmoid(o + b2_ref[...])

    return pl.pallas_call(
        body,
        out_shape=jax.ShapeDtypeStruct((batch, W2.shape[1]), jnp.float32),
    )(pooled, W1, b1.reshape(1, -1), W2, b2.reshape(1, -1))


def kernel(features, table, W1, b1, W2, b2):
    batch, seq = features.shape
    emb = table.shape[1]
    flat_idx = features.reshape(batch * seq)
    pooled_sum = _gather_pool_sc(flat_idx, table, batch, seq, emb)
    return _mlp_tc(pooled_sum, W1, b1, W2, b2, 1.0 / seq)
